# Initial kernel scaffold; baseline (speedup 1.0000x reference)
import jax
import jax.numpy as jnp
from jax.experimental import pallas as pl


def kernel(gt_boxes, anchors, score_labels, confidence_labels):
    raise NotImplementedError("write your pallas kernel here")


# speedup vs baseline: 4.6223x; 4.6223x over previous
import functools

import jax
import jax.numpy as jnp
from jax import lax
from jax.experimental import pallas as pl
from jax.experimental.pallas import tpu as pltpu
from jax.experimental.pallas import tpu_sc as plsc

LOW_THRESH = 0.3
HIGH_THRESH = 0.7

_NW = 32
_LANES = 16


def _matcher_body(an_ref, gt_ref, out_ref, q_ref, gm_ref, *, gp, g, nb, b):
    gx1 = gt_ref[:, 0:1]
    gy1 = gt_ref[:, 1:2]
    gx2 = gt_ref[:, 2:3]
    gy2 = gt_ref[:, 3:4]
    ga = (gx2 - gx1) * (gy2 - gy1)
    for j in range(nb):
        sl = pl.ds(j * b, b)
        ax1 = an_ref[0:1, sl]
        ay1 = an_ref[1:2, sl]
        ax2 = an_ref[2:3, sl]
        ay2 = an_ref[3:4, sl]
        ab = (ax2 - ax1) * (ay2 - ay1)
        w = jnp.maximum(jnp.minimum(gx2, ax2) - jnp.maximum(gx1, ax1), 0.0)
        h = jnp.maximum(jnp.minimum(gy2, ay2) - jnp.maximum(gy1, ay1), 0.0)
        inter = w * h
        q = inter / (ga + ab - inter)
        q_ref[:, sl] = q
        bm = jnp.max(q, axis=1, keepdims=True)
        if j == 0:
            gm_ref[:, 0:1] = bm
        else:
            gm_ref[:, 0:1] = jnp.maximum(gm_ref[:, 0:1], bm)
    gm = gm_ref[:, 0:1]
    giota = lax.broadcasted_iota(jnp.int32, (gp, b), 0)
    for j in range(nb):
        sl = pl.ds(j * b, b)
        q = q_ref[:, sl]
        mv = jnp.max(q, axis=0, keepdims=True)
        am = jnp.min(jnp.where(q == mv, giota, gp), axis=0, keepdims=True)
        restore = jnp.any((q == gm) & (giota < g), axis=0, keepdims=True)
        m = jnp.where(mv < LOW_THRESH, -1, jnp.where(mv < HIGH_THRESH, -2, am))
        m = jnp.where(restore, am, m)
        out_ref[0:1, sl] = m


def _sc_labels_body(m_hbm, tbl_hbm, s_hbm, c_hbm, out_hbm,
                    m_v, tbl_v, s_v, c_v, o_v, *, chunk, g, nc):
    wid = lax.axis_index("s") * nc + lax.axis_index("c")
    base = wid * chunk
    pltpu.sync_copy(m_hbm.at[pl.ds(base, chunk)], m_v)
    pltpu.sync_copy(tbl_hbm, tbl_v)
    pltpu.sync_copy(s_hbm, s_v)
    pltpu.sync_copy(c_hbm, c_v)
    lanes = lax.iota(jnp.int32, _LANES)
    for i in range(chunk // _LANES):
        idx = m_v[pl.ds(i * _LANES, _LANES)]
        cl = jnp.clip(idx, 0, g - 1)
        s = plsc.load_gather(s_v, [cl])
        c = plsc.load_gather(c_v, [cl])
        fg = idx >= 0
        lab = jnp.minimum(jnp.where(fg, 1.0, 0.0), s)
        lab = jnp.where(idx == -1, 0.0, lab)
        lab = jnp.where(idx == -2, -1.0, lab)
        lab = jnp.where(fg & (s < 1.0), -1.0, lab)
        lab = jnp.where(fg & (c == 0), -1.0, lab)
        ob = lanes * 5 + (i * _LANES * 5)
        plsc.store_scatter(o_v, [ob], lab)
        b4 = cl * 4
        for k in range(4):
            bk = plsc.load_gather(tbl_v, [b4 + k])
            plsc.store_scatter(o_v, [ob + (k + 1)], bk)
    pltpu.sync_copy(o_v, out_hbm.at[pl.ds(base * 5, chunk * 5)])


def kernel(gt_boxes, anchors, score_labels, confidence_labels):
    n, g = anchors.shape[0], gt_boxes.shape[0]
    f32 = jnp.float32
    b = 2560
    np_ = -(-n // b) * b
    nb = np_ // b
    gp = -(-g // 8) * 8
    chunk = np_ // _NW

    pad_box = jnp.broadcast_to(
        jnp.array([-2.0, -2.0, -1.0, -1.0], f32)[:, None], (4, np_ - n))
    an_t = jnp.concatenate([anchors.T, pad_box], axis=1)
    an_t = jnp.concatenate([an_t, jnp.zeros((4, np_), f32)], axis=0)
    gt_all = jnp.zeros((gp, 8), f32).at[:g, 0:4].set(gt_boxes)

    matcher = pl.pallas_call(
        functools.partial(_matcher_body, gp=gp, g=g, nb=nb, b=b),
        out_shape=jax.ShapeDtypeStruct((1, np_), jnp.int32),
        scratch_shapes=[
            pltpu.VMEM((gp, np_), f32),
            pltpu.VMEM((gp, 128), f32),
        ],
    )
    matches = matcher(an_t, gt_all).reshape(np_)

    tbl = jnp.pad(gt_boxes.reshape(4 * g), (0, 512 - 4 * g))
    s_pad = jnp.pad(score_labels, (0, 128 - g))
    c_pad = jnp.pad(confidence_labels, (0, 128 - g))

    info = plsc.get_sparse_core_info()
    nc = info.num_cores
    sc_labels = functools.partial(
        pl.kernel,
        mesh=plsc.VectorSubcoreMesh(core_axis_name="c", subcore_axis_name="s"),
        compiler_params=pltpu.CompilerParams(needs_layout_passes=False),
        out_type=jax.ShapeDtypeStruct((np_ * 5,), f32),
        scratch_types=[
            pltpu.VMEM((chunk,), jnp.int32),
            pltpu.VMEM((512,), f32),
            pltpu.VMEM((128,), f32),
            pltpu.VMEM((128,), jnp.int32),
            pltpu.VMEM((chunk * 5,), f32),
        ],
    )(functools.partial(_sc_labels_body, chunk=chunk, g=g, nc=nc))
    out_flat = sc_labels(matches, tbl, s_pad, c_pad)
    return out_flat.reshape(np_, 5)[:n]
